# baseline (speedup 1.0000x reference)
import functools

import jax
import jax.numpy as jnp
from jax import lax
from jax.experimental import pallas as pl
from jax.experimental.pallas import tpu as pltpu
from jax.experimental.pallas import tpu_sc as plsc

B, L, E = 4096, 200, 64
N = B * L
NC, NS = 2, 16
NW = NC * NS
NR = N // 128
RW = NR // NW

_mesh = plsc.VectorSubcoreMesh(core_axis_name="c", subcore_axis_name="s")


@functools.partial(
    pl.kernel,
    out_type=jax.ShapeDtypeStruct((1600, 32, 1024), jnp.float32),
    mesh=_mesh,
    scratch_types=[
        pltpu.VMEM((RW, 128), jnp.int32),
        pltpu.VMEM((2, 128, E), jnp.float32),
        pltpu.VMEM((2, 8, 1, 1024), jnp.float32),
        [pltpu.SemaphoreType.DMA] * 2,
        [pltpu.SemaphoreType.DMA] * 2,
    ],
    compiler_params=pltpu.CompilerParams(
        use_tc_tiling_on_sc=False, needs_layout_passes=False
    ),
)
def _emb(idx_hbm, table_hbm, out_hbm, idx_v, rows_v, tr_v, gsems, ssems):
    wid = lax.axis_index("s") * NC + lax.axis_index("c")
    base_row = wid * RW
    pltpu.sync_copy(idx_hbm.at[pl.ds(base_row, RW)], idx_v)

    def gather(k, p):
        return pltpu.make_async_copy(
            table_hbm.at[idx_v.at[k]],
            rows_v.at[p],
            gsems[p],
        )

    def out_slice(k):
        rg = base_row + k
        lr = rg >> 8
        cb = (rg >> 3) & 31
        s = rg & 7
        r0 = 64 * lr + 8 * s
        return out_hbm.at[pl.ds(r0, 8), pl.ds(cb, 1), :]

    def store(k, p):
        return pltpu.make_async_copy(tr_v.at[p], out_slice(k), ssems[p])

    def wait_store(p):
        pltpu.make_async_copy(
            tr_v.at[p],
            out_hbm.at[pl.ds(0, 8), pl.ds(0, 1), :],
            ssems[p],
        ).wait()

    iota16 = lax.iota(jnp.int32, 16)

    gather(0, 0).start()

    def item(k, p, q):
        @pl.when(k + 1 < RW)
        def _():
            gather(k + 1, q).start()

        @pl.when(k >= 2)
        def _():
            wait_store(p)

        gather(k, p).wait()
        rows_p = rows_v.at[p]
        tr_p = tr_v.at[p]

        def tbody(g, carry):
            lanes = iota16 + 16 * g
            for half in range(4):
                vs = []
                for e in range(16 * half, 16 * half + 16):
                    cvec = jnp.broadcast_to(jnp.int32(e), (16,))
                    vs.append(plsc.load_gather(rows_p, [lanes, cvec]))
                for i, e in enumerate(range(16 * half, 16 * half + 16)):
                    eb, s = e // 8, e % 8
                    tr_p[eb, 0, pl.ds(s * 128 + 16 * g, 16)] = vs[i]
            return carry

        lax.fori_loop(0, 8, tbody, 0)
        store(k, p).start()

    def body(g, carry):
        item(2 * g, 0, 1)
        item(2 * g + 1, 1, 0)
        return carry

    lax.fori_loop(0, RW // 2, body, 0)
    wait_store(0)
    wait_store(1)


def kernel(tok_ids, table):
    idx = (
        tok_ids.T.reshape(25, 8, 32, 128)
        .transpose(0, 2, 1, 3)
        .reshape(NR, 128)
        .astype(jnp.int32)
    )
    out_t = _emb(idx, table)
    out = (
        out_t.reshape(L, 8, 32, 8, 128)
        .transpose(2, 4, 0, 1, 3)
        .reshape(B, L, E)
    )
    return out


# speedup vs baseline: 1.1413x; 1.1413x over previous
import functools

import jax
import jax.numpy as jnp
from jax import lax
from jax.experimental import pallas as pl
from jax.experimental.pallas import tpu as pltpu
from jax.experimental.pallas import tpu_sc as plsc

B, L, E = 4096, 200, 64
N = B * L
NC, NS = 2, 16
NW = NC * NS
NR = N // 128
RW = NR // NW

_mesh = plsc.VectorSubcoreMesh(core_axis_name="c", subcore_axis_name="s")


@functools.partial(
    pl.kernel,
    out_type=jax.ShapeDtypeStruct((1600, 32, 1024), jnp.float32),
    mesh=_mesh,
    scratch_types=[
        pltpu.VMEM((RW, 128), jnp.int32),
        pltpu.VMEM((2, 128, E), jnp.float32),
        pltpu.VMEM((2, 8, 1, 1024), jnp.float32),
        [pltpu.SemaphoreType.DMA] * 2,
        [pltpu.SemaphoreType.DMA] * 2,
    ],
    compiler_params=pltpu.CompilerParams(
        use_tc_tiling_on_sc=False, needs_layout_passes=False
    ),
)
def _emb(idx_hbm, table_hbm, out_hbm, idx_v, rows_v, tr_v, gsems, ssems):
    wid = lax.axis_index("s") * NC + lax.axis_index("c")
    base_row = wid * RW
    pltpu.sync_copy(idx_hbm.at[pl.ds(base_row, RW)], idx_v)

    def gather(k, p):
        return pltpu.make_async_copy(
            table_hbm.at[idx_v.at[k]],
            rows_v.at[p],
            gsems[p],
        )

    def out_slice(k):
        rg = base_row + k
        lr = rg >> 8
        cb = (rg >> 3) & 31
        s = rg & 7
        r0 = 64 * lr + 8 * s
        return out_hbm.at[pl.ds(r0, 8), pl.ds(cb, 1), :]

    def store(k, p):
        return pltpu.make_async_copy(tr_v.at[p], out_slice(k), ssems[p])

    def wait_store(p):
        pltpu.make_async_copy(
            tr_v.at[p],
            out_hbm.at[pl.ds(0, 8), pl.ds(0, 1), :],
            ssems[p],
        ).wait()

    iota16 = lax.iota(jnp.int32, 16)
    z16 = iota16 & 0
    dcol = [(iota16 + d) & 15 for d in range(16)]
    k0c = [dcol[d] >> 3 for d in range(16)]
    k1c = [((iota16 + d) & 7) * 128 + iota16 for d in range(16)]

    gather(0, 0).start()

    def item(k, p, q):
        @pl.when(k + 1 < RW)
        def _():
            gather(k + 1, q).start()

        @pl.when(k >= 2)
        def _():
            wait_store(p)

        gather(k, p).wait()
        rows_p = rows_v.at[p]
        tr_p = tr_v.at[p]

        def tbody(g, carry):
            lanes = iota16 + 16 * g
            for d in range(16):
                i1 = k1c[d] + 16 * g
                for h in range(4):
                    colv = dcol[d] + 16 * h
                    v = plsc.load_gather(rows_p, [lanes, colv])
                    i0 = k0c[d] + 2 * h
                    plsc.store_scatter(tr_p, [i0, z16, i1], v)
            return carry

        lax.fori_loop(0, 8, tbody, 0)
        store(k, p).start()

    def body(g, carry):
        item(2 * g, 0, 1)
        item(2 * g + 1, 1, 0)
        return carry

    lax.fori_loop(0, RW // 2, body, 0)
    wait_store(0)
    wait_store(1)


def kernel(tok_ids, table):
    idx = (
        tok_ids.T.reshape(25, 8, 32, 128)
        .transpose(0, 2, 1, 3)
        .reshape(NR, 128)
        .astype(jnp.int32)
    )
    out_t = _emb(idx, table)
    out = (
        out_t.reshape(L, 8, 32, 8, 128)
        .transpose(2, 4, 0, 1, 3)
        .reshape(B, L, E)
    )
    return out
